# baseline (speedup 1.0000x reference)
import functools
import jax
import jax.numpy as jnp
from jax import lax
from jax.experimental import pallas as pl
from jax.experimental.pallas import tpu as pltpu
from jax.experimental.pallas import tpu_sc as plsc

_R = 128
_C = 1024
_DEPTH = 8


def _sc_body(emb_hbm, out_hbm, emb_v, strip_v, sem):
    Sq, Sv, D = out_hbm.shape
    n_emb = emb_v.shape[0]
    max_pos = (n_emb - 1) // 2
    strip_rows = _C + _R - 1

    ib = lax.axis_index("s")
    jb = lax.axis_index("c")
    i0 = ib * _R
    j0 = jb * _C
    s_lo = (Sq - 1) - (i0 + _R - 1) + j0

    pltpu.sync_copy(emb_hbm, emb_v)

    def build(t, carry):
        u = jnp.clip(s_lo + t - (Sq - 1), -max_pos, max_pos) + max_pos
        for m in range(D // 16):
            strip_v[t, pl.ds(m * 16, 16)] = emb_v[u, pl.ds(m * 16, 16)]
        return carry

    lax.fori_loop(0, strip_rows, build, 0)

    def row_copy(r):
        return pltpu.make_async_copy(
            strip_v.at[pl.ds((_R - 1) - r, _C), :],
            out_hbm.at[i0 + r, pl.ds(j0, _C), :],
            sem,
        )

    def fire(r, carry):
        @pl.when(r >= _DEPTH)
        def _():
            row_copy(r - _DEPTH).wait()

        row_copy(r).start()
        return carry

    lax.fori_loop(0, _R, fire, 0)
    for r in range(_R - _DEPTH, _R):
        row_copy(r).wait()


def kernel(q, v, embeddings):
    Sq = q.shape[1]
    Sv = v.shape[1]
    n_emb, d = embeddings.shape
    mesh = plsc.VectorSubcoreMesh(core_axis_name="c", subcore_axis_name="s")
    run = pl.kernel(
        _sc_body,
        out_type=jax.ShapeDtypeStruct((Sq, Sv, d), embeddings.dtype),
        mesh=mesh,
        scratch_types=[
            pltpu.VMEM((n_emb, d), embeddings.dtype),
            pltpu.VMEM((_C + _R, d), embeddings.dtype),
            pltpu.SemaphoreType.DMA,
        ],
        compiler_params=pltpu.CompilerParams(use_tc_tiling_on_sc=False),
    )
    return run(embeddings)


# speedup vs baseline: 1.3131x; 1.3131x over previous
import jax
import jax.numpy as jnp
from jax import lax
from jax.experimental import pallas as pl
from jax.experimental.pallas import tpu as pltpu
from jax.experimental.pallas import tpu_sc as plsc

_R = 256
_C = 512
_P = 384
_DEPTH = 8


def _sc_body(emb_hbm, out_hbm, emb_v, strip_e, strip_o, sem):
    Sq = out_hbm.shape[0]
    n_emb = emb_v.shape[0]
    max_pos = (n_emb - 1) // 2
    d = emb_v.shape[1]

    wid = lax.axis_index("c") * 16 + lax.axis_index("s")
    ib = wid // 4
    jb = wid % 4
    i0 = ib * _R
    j0 = jb * _C
    s_lo = (Sq - 1) - (i0 + _R - 1) + j0

    pltpu.sync_copy(emb_hbm, emb_v)

    def build(p, carry):
        u0 = jnp.clip(s_lo + 2 * p - (Sq - 1), -max_pos, max_pos) + max_pos
        u1 = jnp.clip(s_lo + 2 * p + 1 - (Sq - 1), -max_pos, max_pos) + max_pos
        u2 = jnp.clip(s_lo + 2 * p + 2 - (Sq - 1), -max_pos, max_pos) + max_pos
        for m in range(d // 16):
            e0 = emb_v[u0, pl.ds(m * 16, 16)]
            e1 = emb_v[u1, pl.ds(m * 16, 16)]
            e2 = emb_v[u2, pl.ds(m * 16, 16)]
            strip_e[p, pl.ds(m * 16, 16)] = e0
            strip_e[p, pl.ds(d + m * 16, 16)] = e1
            strip_o[p, pl.ds(m * 16, 16)] = e1
            strip_o[p, pl.ds(d + m * 16, 16)] = e2
        return carry

    lax.fori_loop(0, _P, build, 0)

    def descr(r, strip):
        dd = (_R - 1) - r
        return pltpu.make_async_copy(
            strip.at[pl.ds(dd >> 1, _C // 2), :],
            out_hbm.at[i0 + r, pl.ds(jb * (_C // 2), _C // 2), :],
            sem,
        )

    def fire(r, carry):
        @pl.when(r >= _DEPTH)
        def _():
            descr(r - _DEPTH, strip_e).wait()

        dd = (_R - 1) - r

        @pl.when(lax.rem(dd, 2) == 0)
        def _():
            descr(r, strip_e).start()

        @pl.when(lax.rem(dd, 2) == 1)
        def _():
            descr(r, strip_o).start()

        return carry

    lax.fori_loop(0, _R, fire, 0)
    for r in range(_R - _DEPTH, _R):
        descr(r, strip_e).wait()


def kernel(q, v, embeddings):
    Sq = q.shape[1]
    Sv = v.shape[1]
    n_emb, d = embeddings.shape
    mesh = plsc.VectorSubcoreMesh(core_axis_name="c", subcore_axis_name="s")
    run = pl.kernel(
        _sc_body,
        out_type=jax.ShapeDtypeStruct((Sq, Sv // 2, 2 * d), embeddings.dtype),
        mesh=mesh,
        scratch_types=[
            pltpu.VMEM((n_emb, d), embeddings.dtype),
            pltpu.VMEM((_P, 2 * d), embeddings.dtype),
            pltpu.VMEM((_P, 2 * d), embeddings.dtype),
            pltpu.SemaphoreType.DMA,
        ],
        compiler_params=pltpu.CompilerParams(use_tc_tiling_on_sc=True),
    )
    return run(embeddings).reshape(Sq, Sv, d)
